# baseline (speedup 1.0000x reference)
import functools

import jax
import jax.numpy as jnp
from jax import lax
from jax.experimental import pallas as pl
from jax.experimental.pallas import tpu as pltpu
from jax.experimental.pallas import tpu_sc as plsc

GAMMA = 12.0
NC, NS, L = 2, 16, 16
NW = NC * NS
B = 4096
NEG = 128
D = 64
RPW = B // NW
NG = NEG // L
NBUF = 4

EBLK = 4096
EQ = 1 << 18
RQ = 256


def _pack32(y):
    lo = lax.bitcast_convert_type(y[:, :32].astype(jnp.bfloat16), jnp.uint16)
    hi = lax.bitcast_convert_type(y[:, 32:].astype(jnp.bfloat16), jnp.uint16)
    u = (lax.convert_element_type(lo, jnp.uint32)
         | (lax.convert_element_type(hi, jnp.uint32) << 16))
    return lax.bitcast_convert_type(u, jnp.int32)


def _repack_body(x0_ref, x1_ref, x2_ref, x3_ref, out_ref):
    out_ref[...] = jnp.concatenate(
        [_pack32(x_ref[...].T) for x_ref in (x0_ref, x1_ref, x2_ref, x3_ref)],
        axis=1)


def _repack(table_t, q, blk):
    nblk = q // blk
    last = pl.cdiv(table_t.shape[1], blk) - 1

    def imap(o):
        return lambda i, m=last: (0, jnp.minimum(i + o, m))

    return pl.pallas_call(
        _repack_body,
        grid=(nblk,),
        in_specs=[pl.BlockSpec((D, blk), imap(k * nblk)) for k in range(4)],
        out_specs=pl.BlockSpec((blk, 2 * D), lambda i: (i, 0)),
        out_shape=jax.ShapeDtypeStruct((q, 2 * D), jnp.int32),
    )(table_t, table_t, table_t, table_t)


def _sc_body(hrow_hbm, rrow_hbm, nrow_hbm, ent_hbm, rel_hbm, out_hbm,
             hrow_v, rrow_v, nrow_v, hr_v, out_v, tail_v, sem, *bufsems):
    wid = lax.axis_index("s") * NC + lax.axis_index("c")
    base = wid * RPW

    pltpu.sync_copy(hrow_hbm.at[pl.ds(base, RPW)], hrow_v)
    pltpu.sync_copy(rrow_hbm.at[pl.ds(base, RPW)], rrow_v)
    pltpu.sync_copy(nrow_hbm.at[pl.ds(base, RPW)], nrow_v)

    iota = lax.iota(jnp.int32, L)
    row_ids = [g * L + iota for g in range(NG)]
    tails = [tail_v.at[j] for j in range(NBUF)]
    sems = list(bufsems)

    pltpu.async_copy(ent_hbm.at[hrow_v], tails[0], sems[0]).wait()
    pltpu.async_copy(rel_hbm.at[rrow_v], tails[1], sems[1]).wait()

    ilv = plsc.PackFormat.INTERLEAVED

    def unpk(u32):
        return plsc.unpack(plsc.bitcast(u32, jnp.bfloat16), format=ilv)

    @pl.loop(0, RPW)
    def _build_hr(b):
        for c in range(D // 2 // L):
            sl = pl.ds(c * L, L)
            he, ho = unpk(tails[0][b, sl])
            re, ro = unpk(tails[1][b, sl])
            packed = plsc.pack(he + re, ho + ro, format=ilv)
            hr_v[b, sl] = plsc.bitcast(packed, jnp.int32)

    def start(row, j):
        pltpu.async_copy(ent_hbm.at[nrow_v.at[row]], tails[j], sems[j])

    def wait(row, j):
        pltpu.make_async_copy(ent_hbm.at[nrow_v.at[row]], tails[j],
                              sems[j]).wait()

    def compute(b, j):
        rows_b = jnp.full((L,), b, jnp.int32)

        def dbody(d, scs):
            cols = jnp.bitwise_and(iota + d, D // 2 - 1)
            hre, hro = unpk(plsc.load_gather(hr_v, [rows_b, cols]))
            out = []
            for s, rid in zip(scs, row_ids):
                te, to = unpk(plsc.load_gather(tails[j], [rid, cols]))
                out.append(s + (jnp.abs(hre - te) + jnp.abs(hro - to)))
            return tuple(out)

        scores = lax.fori_loop(
            0, D // 2, dbody,
            tuple(jnp.zeros((L,), jnp.float32) for _ in range(NG)),
            unroll=2)

        for g in range(NG):
            out_v[b, pl.ds(g * L, L)] = GAMMA - scores[g]

    for j in range(NBUF - 1):
        start(j, j)

    @pl.loop(0, RPW, step=NBUF)
    def _row(i):
        for j in range(NBUF):
            b = i + j
            nxt = jnp.minimum(b + NBUF - 1, RPW - 1)
            start(nxt, (j + NBUF - 1) % NBUF)
            wait(b, j)
            compute(b, j)

    for j in range(NBUF - 1):
        wait(RPW - 1, j)

    pltpu.sync_copy(out_v, out_hbm.at[pl.ds(base, RPW)])


@jax.jit
def _score(hrow, rrow, nrow, ent3, rel3):
    mesh = plsc.VectorSubcoreMesh(core_axis_name="c", subcore_axis_name="s")
    fn = functools.partial(
        pl.kernel,
        out_type=jax.ShapeDtypeStruct((B, NEG), jnp.float32),
        mesh=mesh,
        scratch_types=[
            pltpu.VMEM((RPW,), jnp.int32),
            pltpu.VMEM((RPW,), jnp.int32),
            pltpu.VMEM((RPW, NEG), jnp.int32),
            pltpu.VMEM((RPW, D // 2), jnp.int32),
            pltpu.VMEM((RPW, NEG), jnp.float32),
            pltpu.VMEM((NBUF, NEG, D // 2), jnp.int32),
            pltpu.SemaphoreType.DMA,
            *[pltpu.SemaphoreType.DMA for _ in range(NBUF)],
        ],
        compiler_params=pltpu.CompilerParams(
            use_tc_tiling_on_sc=False, needs_layout_passes=False),
    )(_sc_body)
    return fn(hrow, rrow, nrow, ent3, rel3)


def _flat(idx, q):
    return ((jnp.bitwise_and(idx, q - 1) << 2) | (idx // q)).astype(jnp.int32)


def kernel(positive_sample, negative_sample, entity_embedding,
           relation_embedding):
    ent3 = _repack(entity_embedding.T, EQ, EBLK).reshape(4 * EQ, D // 2)
    rel3 = _repack(relation_embedding.T, RQ, RQ).reshape(4 * RQ, D // 2)

    hidx = positive_sample[:, 0].astype(jnp.int32)
    ridx = positive_sample[:, 1].astype(jnp.int32)
    neg = negative_sample.astype(jnp.int32)

    return _score(_flat(hidx, EQ), _flat(ridx, RQ), _flat(neg, EQ),
                  ent3, rel3)


# speedup vs baseline: 1.1588x; 1.1588x over previous
import functools

import jax
import jax.numpy as jnp
from jax import lax
from jax.experimental import pallas as pl
from jax.experimental.pallas import tpu as pltpu
from jax.experimental.pallas import tpu_sc as plsc

GAMMA = 12.0
NC, NS, L = 2, 16, 16
NW = NC * NS
B = 4096
NEG = 128
D = 64
RPW = B // NW
NG = NEG // L
NBUF = 8

EBLK = 16384
NBLK = 31
EOFF = NBLK * EBLK
ROFF = 512


def _repack_body(lo_ref, hi_ref, out_ref):
    out_ref[...] = jnp.concatenate([lo_ref[...].T, hi_ref[...].T], axis=1)


def _repack(table_t, rows, blk, nblk):
    last = pl.cdiv(table_t.shape[1], blk) - 1
    return pl.pallas_call(
        _repack_body,
        grid=(nblk,),
        in_specs=[
            pl.BlockSpec((D, blk), lambda i: (0, i)),
            pl.BlockSpec((D, blk),
                         lambda i, n=nblk, m=last: (0, jnp.minimum(i + n, m))),
        ],
        out_specs=pl.BlockSpec((blk, 2 * D), lambda i: (i, 0)),
        out_shape=jax.ShapeDtypeStruct((rows, 2 * D), jnp.float32),
    )(table_t, table_t)


def _sc_body(hrow_hbm, rrow_hbm, nrow_hbm, ent_hbm, rel_hbm, out_hbm,
             hrow_v, rrow_v, nrow_v, hr_v, out_v, tail_v, sem, *bufsems):
    wid = lax.axis_index("s") * NC + lax.axis_index("c")
    base = wid * RPW

    pltpu.sync_copy(hrow_hbm.at[pl.ds(base, RPW)], hrow_v)
    pltpu.sync_copy(rrow_hbm.at[pl.ds(base, RPW)], rrow_v)
    pltpu.sync_copy(nrow_hbm.at[pl.ds(base, RPW)], nrow_v)

    iota = lax.iota(jnp.int32, L)
    row_ids = [g * L + iota for g in range(NG)]
    tails = [tail_v.at[j] for j in range(NBUF)]
    sems = list(bufsems)

    pltpu.async_copy(ent_hbm.at[hrow_v], tails[0], sems[0]).wait()
    pltpu.async_copy(rel_hbm.at[rrow_v], tails[1], sems[1]).wait()

    @pl.loop(0, RPW)
    def _build_hr(b):
        for c in range(D // L):
            sl = pl.ds(c * L, L)
            hr_v[b, sl] = tails[0][b, sl] + tails[1][b, sl]

    def start(row, j):
        pltpu.async_copy(ent_hbm.at[nrow_v.at[row]], tails[j], sems[j])

    def wait(row, j):
        pltpu.make_async_copy(ent_hbm.at[nrow_v.at[row]], tails[j],
                              sems[j]).wait()

    def compute(b, j):
        rows_b = jnp.full((L,), b, jnp.int32)

        def dbody(d, scs):
            cols = jnp.bitwise_and(iota + d, D - 1)
            hrd = plsc.load_gather(hr_v, [rows_b, cols])
            return tuple(
                s + jnp.abs(hrd - plsc.load_gather(tails[j], [rid, cols]))
                for s, rid in zip(scs, row_ids))

        scores = lax.fori_loop(
            0, D, dbody,
            tuple(jnp.zeros((L,), jnp.float32) for _ in range(NG)),
            unroll=2)

        for g in range(NG):
            out_v[b, pl.ds(g * L, L)] = GAMMA - scores[g]

    for j in range(NBUF - 1):
        start(j, j)

    @pl.loop(0, RPW, step=NBUF)
    def _row(i):
        for j in range(NBUF):
            b = i + j
            nxt = jnp.minimum(b + NBUF - 1, RPW - 1)
            start(nxt, (j + NBUF - 1) % NBUF)
            wait(b, j)
            compute(b, j)

    for j in range(NBUF - 1):
        wait(RPW - 1, j)

    pltpu.sync_copy(out_v, out_hbm.at[pl.ds(base, RPW)])


@jax.jit
def _score(hrow, rrow, nrow, ent3, rel3):
    mesh = plsc.VectorSubcoreMesh(core_axis_name="c", subcore_axis_name="s")
    fn = functools.partial(
        pl.kernel,
        out_type=jax.ShapeDtypeStruct((B, NEG), jnp.float32),
        mesh=mesh,
        scratch_types=[
            pltpu.VMEM((RPW,), jnp.int32),
            pltpu.VMEM((RPW,), jnp.int32),
            pltpu.VMEM((RPW, NEG), jnp.int32),
            pltpu.VMEM((RPW, D), jnp.float32),
            pltpu.VMEM((RPW, NEG), jnp.float32),
            pltpu.VMEM((NBUF, NEG, D), jnp.float32),
            pltpu.SemaphoreType.DMA,
            *[pltpu.SemaphoreType.DMA for _ in range(NBUF)],
        ],
        compiler_params=pltpu.CompilerParams(
            use_tc_tiling_on_sc=False, needs_layout_passes=False),
    )(_sc_body)
    return fn(hrow, rrow, nrow, ent3, rel3)


def _flat(idx, off):
    return jnp.where(idx < off, 2 * idx, 2 * (idx - off) + 1).astype(jnp.int32)


def kernel(positive_sample, negative_sample, entity_embedding,
           relation_embedding):
    ent3 = _repack(entity_embedding.T, EOFF, EBLK, NBLK).reshape(2 * EOFF, D)
    rel3 = _repack(relation_embedding.T, ROFF, ROFF, 1).reshape(2 * ROFF, D)

    hidx = positive_sample[:, 0].astype(jnp.int32)
    ridx = positive_sample[:, 1].astype(jnp.int32)
    neg = negative_sample.astype(jnp.int32)

    return _score(_flat(hidx, EOFF), _flat(ridx, ROFF), _flat(neg, EOFF),
                  ent3, rel3)
